# baseline (speedup 1.0000x reference)
import jax
import jax.numpy as jnp
from jax.experimental import pallas as pl
from jax.experimental.pallas import tpu as pltpu
from jax.experimental.pallas import tpu_sc as plsc

D_MODEL = 64
SCALE = 8.0
WINDOW = 128
LANES = 16


def kernel(x, table):
    B, S = x.shape
    N = B * S
    idx = x.reshape(1, N).astype(jnp.int32)
    mesh = plsc.VectorSubcoreMesh(core_axis_name="c", subcore_axis_name="s")

    @pl.kernel(
        out_type=jax.ShapeDtypeStruct((N, D_MODEL), jnp.float32),
        mesh=mesh,
        compiler_params=pltpu.CompilerParams(use_tc_tiling_on_sc=False),
    )
    def emb_kernel(tbl_hbm, i_hbm, o_hbm):
        def body(i_vmem, o_vmem):
            pltpu.sync_copy(tbl_hbm.at[i_vmem.at[0]], o_vmem)

            @pl.loop(0, WINDOW)
            def _(r):
                for c in range(0, D_MODEL, LANES):
                    slc = (pl.ds(r, 1), pl.ds(c, LANES))
                    o_vmem.at[*slc][...] = o_vmem.at[*slc][...] * SCALE

        pltpu.emit_pipeline(
            body,
            grid=(N // WINDOW,),
            in_specs=[pl.BlockSpec((1, WINDOW), lambda i: (0, i))],
            out_specs=[pl.BlockSpec((WINDOW, D_MODEL), lambda i: (i, 0))],
            core_axis_name=("c", "s"),
            dimension_semantics=(pltpu.PARALLEL,),
        )(i_hbm, o_hbm)

    out = emb_kernel(table, idx)
    return out.reshape(B, S, D_MODEL)


# speedup vs baseline: 1.7017x; 1.7017x over previous
import jax
import jax.numpy as jnp
from jax.experimental import pallas as pl
from jax.experimental.pallas import tpu as pltpu
from jax.experimental.pallas import tpu_sc as plsc

D_MODEL = 64
SCALE = 8.0
WINDOW = 128
TILE_C = 2048


def _scale_widen(table_t):
    V = table_t.shape[1]

    def body(in_ref, out_ref):
        t = jnp.transpose(in_ref[...]) * SCALE
        out_ref[...] = jnp.concatenate([t, t], axis=1)

    grid = (V + TILE_C - 1) // TILE_C
    return pl.pallas_call(
        body,
        grid=(grid,),
        in_specs=[pl.BlockSpec((D_MODEL, TILE_C), lambda i: (0, i))],
        out_specs=pl.BlockSpec((TILE_C, 128), lambda i: (i, 0)),
        out_shape=jax.ShapeDtypeStruct((V, 128), jnp.float32),
    )(table_t)


def kernel(x, table):
    B, S = x.shape
    N = B * S
    idx = x.reshape(1, N).astype(jnp.int32)

    wide = _scale_widen(table.T)

    mesh = plsc.VectorSubcoreMesh(core_axis_name="c", subcore_axis_name="s")

    @pl.kernel(
        out_type=jax.ShapeDtypeStruct((N, 128), jnp.float32),
        mesh=mesh,
        compiler_params=pltpu.CompilerParams(use_tc_tiling_on_sc=False),
    )
    def emb_kernel(tbl_hbm, i_hbm, o_hbm):
        def body(i_vmem, o_vmem):
            pltpu.sync_copy(tbl_hbm.at[i_vmem.at[0]], o_vmem)

        pltpu.emit_pipeline(
            body,
            grid=(N // WINDOW,),
            in_specs=[pl.BlockSpec((1, WINDOW), lambda i: (0, i))],
            out_specs=[pl.BlockSpec((WINDOW, 128), lambda i: (i, 0))],
            core_axis_name=("c", "s"),
            dimension_semantics=(pltpu.PARALLEL,),
        )(i_hbm, o_hbm)

    out = emb_kernel(wide, idx)
    return out[:, :D_MODEL].reshape(B, S, D_MODEL)
